# baseline (speedup 1.0000x reference)
import jax
import jax.numpy as jnp
from jax import lax
from jax.experimental import pallas as pl
from jax.experimental.pallas import tpu as pltpu
from jax.experimental.pallas import tpu_sc as plsc

NC, NS, L = 2, 16, 16
NW = NC * NS
BATCH, SEQ, DIM = 4096, 200, 64
PD = 128
RPW = BATCH // NW
C0, C1 = 104, 96
NG = 2


def _body(xf_hbm, wt_hbm, pt_hbm, out_hbm, idx_v, pos_lin, gbuf, obuf,
          gsem, osem):
    wid = lax.axis_index("s") * NC + lax.axis_index("c")
    base = pl.multiple_of(wid * RPW * SEQ, 8)
    bbase = wid * RPW

    colv = [c * L + lax.iota(jnp.int32, 16) for c in range(DIM // L)]

    pltpu.sync_copy(pt_hbm, obuf)

    @pl.loop(0, SEQ)
    def _lin(s):
        rowv = jnp.full((16,), s, jnp.int32)
        o = pl.multiple_of(s * DIM, DIM)
        for c in range(DIM // L):
            v = plsc.load_gather(obuf, [rowv, colv[c]])
            pos_lin[pl.ds(o + c * L, L)] = v

    pltpu.sync_copy(xf_hbm.at[pl.ds(base, RPW * SEQ)], idx_v)

    def fire_gather(q, r):
        o = pl.multiple_of(r * SEQ, 8)
        pltpu.async_copy(wt_hbm.at[idx_v.at[pl.ds(o, C0)]],
                         gbuf.at[pl.ds(q * SEQ, C0)], gsem.at[q])
        pltpu.async_copy(wt_hbm.at[idx_v.at[pl.ds(o + C0, C1)]],
                         gbuf.at[pl.ds(q * SEQ + C0, C1)], gsem.at[q])

    def wait_gather(q):
        pltpu.make_async_copy(wt_hbm.at[pl.ds(0, SEQ)],
                              gbuf.at[pl.ds(q * SEQ, SEQ)],
                              gsem.at[q]).wait()

    def move_add(q):
        @pl.loop(0, SEQ, unroll=2)
        def _seq(s):
            rowg = jnp.full((16,), q * SEQ + s, jnp.int32)
            rowo = jnp.full((16,), s, jnp.int32)
            o = pl.multiple_of(s * DIM, DIM)
            for c in range(DIM // L):
                wv = plsc.load_gather(gbuf, [rowg, colv[c]])
                pv = pos_lin[pl.ds(o + c * L, L)]
                plsc.store_scatter(obuf, [rowo, colv[c]], wv + pv)

    def fire_out(r):
        pltpu.async_copy(obuf, out_hbm.at[bbase + r], osem.at[0])

    def wait_out():
        pltpu.make_async_copy(obuf, out_hbm.at[0], osem.at[0]).wait()

    for q in range(NG):
        fire_gather(q, q)

    wait_gather(0)
    move_add(0)
    fire_out(0)
    fire_gather(0, NG)
    wait_gather(1)
    wait_out()
    move_add(1)
    fire_out(1)
    fire_gather(1, NG + 1)

    @pl.loop(NG, RPW - NG, step=NG)
    def _ring(g):
        for q in range(NG):
            wait_gather(q)
            wait_out()
            move_add(q)
            fire_out(g + q)
            fire_gather(q, g + NG + q)

    for q in range(NG):
        wait_gather(q)
        wait_out()
        move_add(q)
        fire_out(RPW - NG + q)
    wait_out()


def kernel(x, word_table, pos_table):
    xf = jnp.reshape(x.astype(jnp.int32), (-1,))
    wtp = jnp.pad(word_table, ((0, 0), (0, PD - DIM)))
    mesh = plsc.VectorSubcoreMesh(core_axis_name="c", subcore_axis_name="s")
    f = pl.kernel(
        _body,
        out_type=jax.ShapeDtypeStruct((BATCH, SEQ, DIM), jnp.float32),
        mesh=mesh,
        scratch_types=[
            pltpu.VMEM((RPW * SEQ,), jnp.int32),
            pltpu.VMEM((SEQ * DIM,), jnp.float32),
            pltpu.VMEM((NG * SEQ, PD), jnp.float32),
            pltpu.VMEM((SEQ, DIM), jnp.float32),
            pltpu.SemaphoreType.DMA((NG,)),
            pltpu.SemaphoreType.DMA((1,)),
        ],
        compiler_params=pltpu.CompilerParams(use_tc_tiling_on_sc=True,
                                             needs_layout_passes=False),
    )
    return f(xf, wtp, pos_table)


# speedup vs baseline: 1.1501x; 1.1501x over previous
import jax
import jax.numpy as jnp
from jax import lax
from jax.experimental import pallas as pl
from jax.experimental.pallas import tpu as pltpu
from jax.experimental.pallas import tpu_sc as plsc

NC, NS, L = 2, 16, 16
NW = NC * NS
BATCH, SEQ, DIM = 4096, 200, 64
RPW = BATCH // NW
C0, C1 = 104, 96
NBUF = 4


def _body(x_hbm, wt_hbm, pt_hbm, out_hbm, idx_v, pos_v, buf, gsem, osem):
    wid = lax.axis_index("s") * NC + lax.axis_index("c")
    base = wid * RPW
    pltpu.sync_copy(pt_hbm, pos_v)
    pltpu.sync_copy(x_hbm.at[pl.ds(base, RPW)], idx_v)

    def fire_gather(j, r):
        pltpu.async_copy(wt_hbm.at[idx_v.at[r, pl.ds(0, C0)]],
                         buf.at[j, pl.ds(0, C0)], gsem.at[j])
        pltpu.async_copy(wt_hbm.at[idx_v.at[r, pl.ds(C0, C1)]],
                         buf.at[j, pl.ds(C0, C1)], gsem.at[j])

    def wait_gather(j):
        pltpu.make_async_copy(wt_hbm.at[pl.ds(0, SEQ)], buf.at[j],
                              gsem.at[j]).wait()

    def pos_add(j):
        @pl.loop(0, SEQ, unroll=8)
        def _seq(s):
            for c in range(DIM // L):
                plsc.addupdate(buf.at[j, s, pl.ds(c * L, L)],
                               pos_v[s, pl.ds(c * L, L)])

    def fire_out(j, r):
        pltpu.async_copy(buf.at[j], out_hbm.at[base + r], osem.at[j])

    def wait_out(j):
        pltpu.make_async_copy(buf.at[j], out_hbm.at[0], osem.at[j]).wait()

    for j in range(NBUF):
        fire_gather(j, j)

    @pl.loop(0, RPW - NBUF, step=NBUF)
    def _ring(g):
        for j in range(NBUF):
            wait_gather(j)
            pos_add(j)
            fire_out(j, g + j)
        for j in range(NBUF):
            wait_out(j)
            fire_gather(j, g + NBUF + j)

    for j in range(NBUF):
        wait_gather(j)
        pos_add(j)
        fire_out(j, RPW - NBUF + j)
    for j in range(NBUF):
        wait_out(j)


def kernel(x, word_table, pos_table):
    x = x.astype(jnp.int32)
    mesh = plsc.VectorSubcoreMesh(core_axis_name="c", subcore_axis_name="s")
    f = pl.kernel(
        _body,
        out_type=jax.ShapeDtypeStruct((BATCH, SEQ, DIM), jnp.float32),
        mesh=mesh,
        scratch_types=[
            pltpu.VMEM((RPW, SEQ), jnp.int32),
            pltpu.VMEM((SEQ, DIM), jnp.float32),
            pltpu.VMEM((NBUF, SEQ, DIM), jnp.float32),
            pltpu.SemaphoreType.DMA((NBUF,)),
            pltpu.SemaphoreType.DMA((NBUF,)),
        ],
        compiler_params=pltpu.CompilerParams(use_tc_tiling_on_sc=False),
    )
    return f(x, word_table, pos_table)
